# baseline (speedup 1.0000x reference)
import functools

import jax
import jax.numpy as jnp
from jax import lax
from jax.experimental import pallas as pl
from jax.experimental.pallas import tpu as pltpu
from jax.experimental.pallas import tpu_sc as plsc

N = 10000
E = 320000
OUT = 64
NPAD = 10240
NC = 2
NS = 16
NW = NC * NS
CHUNK = 64
NCH = E // CHUNK
CBASE = NCH // NW
CEXTRA = NCH - CBASE * NW
CMAX = CBASE + 1
RA = 624
RREM_OFF = RA * NS
RREM = N - RREM_OFF
HPT = NPAD // NS
HW = 16

_mesh = lambda: plsc.VectorSubcoreMesh(core_axis_name="c", subcore_axis_name="s")


def _tile_chunks(c, s):
    w = c * NS + s
    start = w * CBASE + jnp.minimum(w, CEXTRA)
    nc = jnp.where(w < CEXTRA, CBASE + 1, CBASE)
    return start, nc


@functools.partial(
    pl.kernel,
    mesh=_mesh(),
    out_type=jax.ShapeDtypeStruct((NC, NPAD, HW), jnp.float32),
    scratch_types=[
        pltpu.VMEM((CHUNK, HW), jnp.float32),
        pltpu.VMEM((CMAX, CHUNK), jnp.int32),
        pltpu.VMEM_SHARED((NPAD, HW), jnp.float32),
        pltpu.SemaphoreType.DMA,
        pltpu.SemaphoreType.DMA,
    ],
)
def _hist(dst_hbm, ones_hbm, out_hbm, ones_v, didx, accum, dsem, ssem):
    c = lax.axis_index("c")
    s = lax.axis_index("s")
    start, nc = _tile_chunks(c, s)

    def _ld(k, carry):
        pltpu.async_copy(dst_hbm.at[pl.ds((start + k) * CHUNK, CHUNK)],
                         didx.at[k], dsem)
        return carry

    lax.fori_loop(0, nc, _ld, 0)
    pltpu.sync_copy(ones_hbm.at[pl.ds(0, CHUNK)], ones_v)
    pltpu.sync_copy(ones_hbm, accum.at[pl.ds(s * HPT, HPT)])

    def _lw(k, carry):
        pltpu.make_async_copy(dst_hbm.at[pl.ds(0, CHUNK)], didx.at[0],
                              dsem).wait()
        return carry

    lax.fori_loop(0, nc, _lw, 0)
    plsc.subcore_barrier()

    def _sc(j, carry):
        pltpu.async_copy(ones_v, accum.at[didx.at[j]], ssem, add=True)
        return carry

    lax.fori_loop(0, nc, _sc, 0)

    def _sw(j, carry):
        pltpu.make_async_copy(ones_v, accum.at[didx.at[0]], ssem).wait()
        return carry

    lax.fori_loop(0, nc, _sw, 0)
    plsc.subcore_barrier()
    pltpu.sync_copy(accum.at[pl.ds(s * HPT, HPT)],
                    out_hbm.at[c, pl.ds(s * HPT, HPT)])


ACH = 128
ANCH = E // ACH
ACBASE = ANCH // NW
ACEXTRA = ANCH - ACBASE * NW
ACMAX = ACBASE + 1


def _agg_tile_chunks(c, s):
    w = c * NS + s
    start = w * ACBASE + jnp.minimum(w, ACEXTRA)
    nc = jnp.where(w < ACEXTRA, ACBASE + 1, ACBASE)
    return start, nc


def _make_agg(C):
    @functools.partial(
        pl.kernel,
        mesh=_mesh(),
        out_type=jax.ShapeDtypeStruct((NC, N, C), jnp.float32),
        scratch_types=[
            pltpu.VMEM((ACMAX * ACH,), jnp.int32),
            pltpu.VMEM((6, ACH), jnp.int32),
            pltpu.VMEM((2, ACH, C), jnp.float32),
            pltpu.VMEM_SHARED((N, C), jnp.float32),
            pltpu.SemaphoreType.DMA,
            pltpu.SemaphoreType.DMA,
            pltpu.SemaphoreType.DMA,
            pltpu.SemaphoreType.DMA,
        ],
    )
    def agg(src_hbm, dst_hbm, y_hbm, out_hbm,
            sidx, didx, rows, accum, dsem, gsem, esem0, esem1):
        c = lax.axis_index("c")
        s = lax.axis_index("s")
        start, nc = _agg_tile_chunks(c, s)

        pltpu.async_copy(src_hbm.at[pl.ds(start * ACH, ACBASE * ACH)],
                         sidx.at[pl.ds(0, ACBASE * ACH)], dsem)

        @pl.when(nc == ACMAX)
        def _():
            pltpu.async_copy(
                src_hbm.at[pl.ds((start + ACBASE) * ACH, ACH)],
                sidx.at[pl.ds(ACBASE * ACH, ACH)], dsem)

        esems = (esem0, esem1)

        def _dld(j, slot, sem):
            pltpu.async_copy(dst_hbm.at[pl.ds((start + j) * ACH, ACH)],
                             didx.at[slot], sem)

        _dld(0, 0, esem0)
        _dld(1, 1, esem1)

        pltpu.sync_copy(y_hbm.at[pl.ds(s * RA, RA)],
                        accum.at[pl.ds(s * RA, RA)])

        @pl.when(s == NS - 1)
        def _():
            pltpu.sync_copy(y_hbm.at[pl.ds(RREM_OFF, RREM)],
                            accum.at[pl.ds(RREM_OFF, RREM)])

        pltpu.make_async_copy(src_hbm.at[pl.ds(0, ACBASE * ACH)],
                              sidx.at[pl.ds(0, ACBASE * ACH)], dsem).wait()

        @pl.when(nc == ACMAX)
        def _():
            pltpu.make_async_copy(src_hbm.at[pl.ds(0, ACH)],
                                  sidx.at[pl.ds(0, ACH)], dsem).wait()

        plsc.subcore_barrier()

        def _gather(j, buf):
            pltpu.async_copy(
                y_hbm.at[sidx.at[pl.ds(j * ACH, ACH)]],
                rows.at[buf], gsem)

        _gather(0, 0)

        NP = (ACMAX + 5) // 6

        def body(p, carry):
            for k in range(6):
                j = 6 * p + k

                @pl.when(j < nc)
                def _():
                    pltpu.make_async_copy(
                        y_hbm.at[sidx.at[pl.ds(0, ACH)]],
                        rows.at[k % 2], gsem).wait()
                    pltpu.make_async_copy(dst_hbm.at[pl.ds(0, ACH)],
                                          didx.at[k], esems[k % 2]).wait()

                    @pl.when(j + 1 < nc)
                    def _():
                        _gather(j + 1, (k + 1) % 2)

                    @pl.when(j + 2 < nc)
                    def _():
                        _dld(j + 2, (k + 2) % 6, esems[k % 2])

                    pltpu.sync_copy(rows.at[k % 2], accum.at[didx.at[k]],
                                    add=True)
            return carry

        lax.fori_loop(0, NP, body, 0)
        plsc.subcore_barrier()
        pltpu.sync_copy(accum.at[pl.ds(s * RA, RA)],
                        out_hbm.at[c, pl.ds(s * RA, RA)])

        @pl.when(s == NS - 1)
        def _():
            pltpu.sync_copy(accum.at[pl.ds(RREM_OFF, RREM)],
                            out_hbm.at[c, pl.ds(RREM_OFF, RREM)])

    return agg


_agg128 = _make_agg(128)


BR = 1000
GRID = N // BR


def _dinv_blk(d_ref):
    deg = d_ref[0, :, 0:1] + d_ref[1, :, 0:1] - 1.0
    return lax.rsqrt(deg)


def _mm1_body(x_ref, w_ref, d_ref, o_ref):
    y = jnp.dot(x_ref[...], w_ref[...], preferred_element_type=jnp.float32)
    o_ref[...] = y * _dinv_blk(d_ref)


def _mm1(x, W1, degp):
    return pl.pallas_call(
        _mm1_body,
        grid=(GRID,),
        in_specs=[
            pl.BlockSpec((BR, 128), lambda i: (i, 0)),
            pl.BlockSpec((128, 128), lambda i: (0, 0)),
            pl.BlockSpec((NC, BR, HW), lambda i: (0, i, 0)),
        ],
        out_specs=pl.BlockSpec((BR, 128), lambda i: (i, 0)),
        out_shape=jax.ShapeDtypeStruct((N, 128), jnp.float32),
    )(x, W1, degp)


def _mm2_body(p_ref, y1_ref, d_ref, b_ref, w_ref, o_ref):
    dinv = _dinv_blk(d_ref)
    y1 = y1_ref[...]
    agg = p_ref[0] + p_ref[1] - y1
    h = jnp.maximum(agg * dinv + b_ref[...][None, :], 0.0)
    o_ref[...] = jnp.dot(h, w_ref[...], preferred_element_type=jnp.float32) * dinv


def _mm2(p1, y1, degp, b1, W2):
    return pl.pallas_call(
        _mm2_body,
        grid=(GRID,),
        in_specs=[
            pl.BlockSpec((NC, BR, 128), lambda i: (0, i, 0)),
            pl.BlockSpec((BR, 128), lambda i: (i, 0)),
            pl.BlockSpec((NC, BR, HW), lambda i: (0, i, 0)),
            pl.BlockSpec((128,), lambda i: (0,)),
            pl.BlockSpec((128, 128), lambda i: (0, 0)),
        ],
        out_specs=pl.BlockSpec((BR, 128), lambda i: (i, 0)),
        out_shape=jax.ShapeDtypeStruct((N, 128), jnp.float32),
    )(p1, y1, degp, b1, W2)


def _final_body(p_ref, y2_ref, d_ref, b_ref, o_ref):
    dinv = _dinv_blk(d_ref)
    o = (p_ref[0, :, :64] + p_ref[1, :, :64] - y2_ref[:, :64]) * dinv \
        + b_ref[...][None, :]
    m = jnp.max(o, axis=1, keepdims=True)
    z = o - m
    o_ref[...] = z - jnp.log(jnp.sum(jnp.exp(z), axis=1, keepdims=True))


def _final(p2, y2, degp, b2):
    return pl.pallas_call(
        _final_body,
        grid=(GRID,),
        in_specs=[
            pl.BlockSpec((NC, BR, 128), lambda i: (0, i, 0)),
            pl.BlockSpec((BR, 128), lambda i: (i, 0)),
            pl.BlockSpec((NC, BR, HW), lambda i: (0, i, 0)),
            pl.BlockSpec((64,), lambda i: (0,)),
        ],
        out_specs=pl.BlockSpec((BR, 64), lambda i: (i, 0)),
        out_shape=jax.ShapeDtypeStruct((N, 64), jnp.float32),
    )(p2, y2, degp, b2)


def kernel(x, edge_index, W1, b1, W2, b2):
    ei = edge_index.astype(jnp.int32)
    src = ei[0]
    dst = ei[1]
    ones = jnp.ones((HPT, HW), jnp.float32)
    W2p = jnp.pad(W2, ((0, 0), (0, 128 - OUT)))
    degp = _hist(dst, ones)
    y1 = _mm1(x, W1, degp)
    p1 = _agg128(src, dst, y1)
    y2 = _mm2(p1, y1, degp, b1, W2p)
    p2 = _agg128(src, dst, y2)
    return _final(p2, y2, degp, b2)


# speedup vs baseline: 1.0037x; 1.0037x over previous
import functools

import jax
import jax.numpy as jnp
from jax import lax
from jax.experimental import pallas as pl
from jax.experimental.pallas import tpu as pltpu
from jax.experimental.pallas import tpu_sc as plsc

N = 10000
E = 320000
OUT = 64
NPAD = 10240
NC = 2
NS = 16
NW = NC * NS
CHUNK = 64
NCH = E // CHUNK
CBASE = NCH // NW
CEXTRA = NCH - CBASE * NW
CMAX = CBASE + 1
RA = 624
RREM_OFF = RA * NS
RREM = N - RREM_OFF
HPT = NPAD // NS
HW = 16

_mesh = lambda: plsc.VectorSubcoreMesh(core_axis_name="c", subcore_axis_name="s")


def _tile_chunks(c, s):
    w = c * NS + s
    start = w * CBASE + jnp.minimum(w, CEXTRA)
    nc = jnp.where(w < CEXTRA, CBASE + 1, CBASE)
    return start, nc


@functools.partial(
    pl.kernel,
    mesh=_mesh(),
    out_type=jax.ShapeDtypeStruct((NC, NPAD, HW), jnp.float32),
    scratch_types=[
        pltpu.VMEM((CHUNK, HW), jnp.float32),
        pltpu.VMEM((CMAX, CHUNK), jnp.int32),
        pltpu.VMEM_SHARED((NPAD, HW), jnp.float32),
        pltpu.SemaphoreType.DMA,
        pltpu.SemaphoreType.DMA,
    ],
)
def _hist(dst_hbm, ones_hbm, out_hbm, ones_v, didx, accum, dsem, ssem):
    c = lax.axis_index("c")
    s = lax.axis_index("s")
    start, nc = _tile_chunks(c, s)

    def _ld(k, carry):
        pltpu.async_copy(dst_hbm.at[pl.ds((start + k) * CHUNK, CHUNK)],
                         didx.at[k], dsem)
        return carry

    lax.fori_loop(0, nc, _ld, 0)
    pltpu.sync_copy(ones_hbm.at[pl.ds(0, CHUNK)], ones_v)
    pltpu.sync_copy(ones_hbm, accum.at[pl.ds(s * HPT, HPT)])

    def _lw(k, carry):
        pltpu.make_async_copy(dst_hbm.at[pl.ds(0, CHUNK)], didx.at[0],
                              dsem).wait()
        return carry

    lax.fori_loop(0, nc, _lw, 0)
    plsc.subcore_barrier()

    def _sc(j, carry):
        pltpu.async_copy(ones_v, accum.at[didx.at[j]], ssem, add=True)
        return carry

    lax.fori_loop(0, nc, _sc, 0)

    def _sw(j, carry):
        pltpu.make_async_copy(ones_v, accum.at[didx.at[0]], ssem).wait()
        return carry

    lax.fori_loop(0, nc, _sw, 0)
    plsc.subcore_barrier()
    pltpu.sync_copy(accum.at[pl.ds(s * HPT, HPT)],
                    out_hbm.at[c, pl.ds(s * HPT, HPT)])


ACH = 128
ANCH = E // ACH
ACBASE = ANCH // NW
ACEXTRA = ANCH - ACBASE * NW
ACMAX = ACBASE + 1


def _agg_tile_chunks(c, s):
    w = c * NS + s
    start = w * ACBASE + jnp.minimum(w, ACEXTRA)
    nc = jnp.where(w < ACEXTRA, ACBASE + 1, ACBASE)
    return start, nc


def _make_agg(C):
    @functools.partial(
        pl.kernel,
        mesh=_mesh(),
        out_type=jax.ShapeDtypeStruct((NC, N, C), jnp.float32),
        scratch_types=[
            pltpu.VMEM((ACMAX * ACH,), jnp.int32),
            pltpu.VMEM((6, ACH), jnp.int32),
            pltpu.VMEM((2, ACH, C), jnp.float32),
            pltpu.VMEM_SHARED((N, C), jnp.float32),
            pltpu.SemaphoreType.DMA,
            pltpu.SemaphoreType.DMA,
            pltpu.SemaphoreType.DMA,
            pltpu.SemaphoreType.DMA,
        ],
    )
    def agg(src_hbm, dst_hbm, y_hbm, z_hbm, out_hbm,
            sidx, didx, rows, accum, dsem, gsem, esem0, esem1):
        c = lax.axis_index("c")
        s = lax.axis_index("s")
        start, nc = _agg_tile_chunks(c, s)

        pltpu.async_copy(src_hbm.at[pl.ds(start * ACH, ACBASE * ACH)],
                         sidx.at[pl.ds(0, ACBASE * ACH)], dsem)

        @pl.when(nc == ACMAX)
        def _():
            pltpu.async_copy(
                src_hbm.at[pl.ds((start + ACBASE) * ACH, ACH)],
                sidx.at[pl.ds(ACBASE * ACH, ACH)], dsem)

        esems = (esem0, esem1)

        def _dld(j, slot, sem):
            pltpu.async_copy(dst_hbm.at[pl.ds((start + j) * ACH, ACH)],
                             didx.at[slot], sem)

        _dld(0, 0, esem0)
        _dld(1, 1, esem1)

        @pl.when(c == 0)
        def _():
            pltpu.sync_copy(y_hbm.at[pl.ds(s * RA, RA)],
                            accum.at[pl.ds(s * RA, RA)])

            @pl.when(s == NS - 1)
            def _():
                pltpu.sync_copy(y_hbm.at[pl.ds(RREM_OFF, RREM)],
                                accum.at[pl.ds(RREM_OFF, RREM)])

        @pl.when(c == 1)
        def _():
            pltpu.sync_copy(z_hbm.at[pl.ds(s * RA, RA)],
                            accum.at[pl.ds(s * RA, RA)])

            @pl.when(s == NS - 1)
            def _():
                pltpu.sync_copy(z_hbm.at[pl.ds(RREM_OFF, RREM)],
                                accum.at[pl.ds(RREM_OFF, RREM)])

        pltpu.make_async_copy(src_hbm.at[pl.ds(0, ACBASE * ACH)],
                              sidx.at[pl.ds(0, ACBASE * ACH)], dsem).wait()

        @pl.when(nc == ACMAX)
        def _():
            pltpu.make_async_copy(src_hbm.at[pl.ds(0, ACH)],
                                  sidx.at[pl.ds(0, ACH)], dsem).wait()

        plsc.subcore_barrier()

        def _gather(j, buf):
            pltpu.async_copy(
                y_hbm.at[sidx.at[pl.ds(j * ACH, ACH)]],
                rows.at[buf], gsem)

        _gather(0, 0)

        NP = (ACMAX + 5) // 6

        def body(p, carry):
            for k in range(6):
                j = 6 * p + k

                @pl.when(j < nc)
                def _():
                    pltpu.make_async_copy(
                        y_hbm.at[sidx.at[pl.ds(0, ACH)]],
                        rows.at[k % 2], gsem).wait()
                    pltpu.make_async_copy(dst_hbm.at[pl.ds(0, ACH)],
                                          didx.at[k], esems[k % 2]).wait()

                    @pl.when(j + 1 < nc)
                    def _():
                        _gather(j + 1, (k + 1) % 2)

                    @pl.when(j + 2 < nc)
                    def _():
                        _dld(j + 2, (k + 2) % 6, esems[k % 2])

                    pltpu.sync_copy(rows.at[k % 2], accum.at[didx.at[k]],
                                    add=True)
            return carry

        lax.fori_loop(0, NP, body, 0)
        plsc.subcore_barrier()
        pltpu.sync_copy(accum.at[pl.ds(s * RA, RA)],
                        out_hbm.at[c, pl.ds(s * RA, RA)])

        @pl.when(s == NS - 1)
        def _():
            pltpu.sync_copy(accum.at[pl.ds(RREM_OFF, RREM)],
                            out_hbm.at[c, pl.ds(RREM_OFF, RREM)])

    return agg


_agg128 = _make_agg(128)


BR = 1000
GRID = N // BR


def _dinv_blk(d_ref):
    deg = d_ref[0, :, 0:1] + d_ref[1, :, 0:1] - 1.0
    return lax.rsqrt(deg)


def _mm1_body(x_ref, w_ref, d_ref, o_ref):
    y = jnp.dot(x_ref[...], w_ref[...], preferred_element_type=jnp.float32)
    o_ref[...] = y * _dinv_blk(d_ref)


def _mm1(x, W1, degp):
    return pl.pallas_call(
        _mm1_body,
        grid=(GRID,),
        in_specs=[
            pl.BlockSpec((BR, 128), lambda i: (i, 0)),
            pl.BlockSpec((128, 128), lambda i: (0, 0)),
            pl.BlockSpec((NC, BR, HW), lambda i: (0, i, 0)),
        ],
        out_specs=pl.BlockSpec((BR, 128), lambda i: (i, 0)),
        out_shape=jax.ShapeDtypeStruct((N, 128), jnp.float32),
    )(x, W1, degp)


def _mm2_body(p_ref, d_ref, b_ref, w_ref, o_ref):
    dinv = _dinv_blk(d_ref)
    agg = p_ref[0] + p_ref[1]
    h = jnp.maximum(agg * dinv + b_ref[...][None, :], 0.0)
    o_ref[...] = jnp.dot(h, w_ref[...], preferred_element_type=jnp.float32) * dinv


def _mm2(p1, degp, b1, W2):
    return pl.pallas_call(
        _mm2_body,
        grid=(GRID,),
        in_specs=[
            pl.BlockSpec((NC, BR, 128), lambda i: (0, i, 0)),
            pl.BlockSpec((NC, BR, HW), lambda i: (0, i, 0)),
            pl.BlockSpec((128,), lambda i: (0,)),
            pl.BlockSpec((128, 128), lambda i: (0, 0)),
        ],
        out_specs=pl.BlockSpec((BR, 128), lambda i: (i, 0)),
        out_shape=jax.ShapeDtypeStruct((N, 128), jnp.float32),
    )(p1, degp, b1, W2)


def _final_body(p_ref, d_ref, b_ref, o_ref):
    dinv = _dinv_blk(d_ref)
    o = (p_ref[0, :, :64] + p_ref[1, :, :64]) * dinv + b_ref[...][None, :]
    m = jnp.max(o, axis=1, keepdims=True)
    z = o - m
    o_ref[...] = z - jnp.log(jnp.sum(jnp.exp(z), axis=1, keepdims=True))


def _final(p2, degp, b2):
    return pl.pallas_call(
        _final_body,
        grid=(GRID,),
        in_specs=[
            pl.BlockSpec((NC, BR, 128), lambda i: (0, i, 0)),
            pl.BlockSpec((NC, BR, HW), lambda i: (0, i, 0)),
            pl.BlockSpec((64,), lambda i: (0,)),
        ],
        out_specs=pl.BlockSpec((BR, 64), lambda i: (i, 0)),
        out_shape=jax.ShapeDtypeStruct((N, 64), jnp.float32),
    )(p2, degp, b2)


def kernel(x, edge_index, W1, b1, W2, b2):
    ei = edge_index.astype(jnp.int32)
    src = ei[0]
    dst = ei[1]
    ones = jnp.ones((HPT, HW), jnp.float32)
    W2p = jnp.pad(W2, ((0, 0), (0, 128 - OUT)))
    zeros = jnp.zeros((N, 128), jnp.float32)
    degp = _hist(dst, ones)
    y1 = _mm1(x, W1, degp)
    p1 = _agg128(src, dst, y1, zeros)
    y2 = _mm2(p1, degp, b1, W2p)
    p2 = _agg128(src, dst, y2, zeros)
    return _final(p2, degp, b2)
